# Initial kernel scaffold; baseline (speedup 1.0000x reference)
import jax
import jax.numpy as jnp
from jax.experimental import pallas as pl


def kernel(x, edge_index, W, b):
    raise NotImplementedError("write your pallas kernel here")


# speedup vs baseline: 4.0740x; 4.0740x over previous
import functools

import jax
import jax.numpy as jnp
from jax import lax
from jax.experimental import pallas as pl
from jax.experimental.pallas import tpu as pltpu
from jax.experimental.pallas import tpu_sc as plsc

N_NODES = 10000
N_EDGES = 320000
D = 128

NC = 2
NS = 16
CHUNK = 128
CHUNKS_PER_TILE = 79
E_TILE = CHUNK * CHUNKS_PER_TILE
E_PAD = E_TILE * NC * NS
N_PAD = 10240
ROWS_PER_TILE = N_PAD // NS


def _sc_aggregate(x, src_p, dst_p):
  mesh = plsc.VectorSubcoreMesh(core_axis_name="c", subcore_axis_name="s")

  @functools.partial(
      pl.kernel,
      mesh=mesh,
      out_type=jax.ShapeDtypeStruct((NC, N_PAD, D), jnp.float32),
      scratch_types=[
          pltpu.VMEM((CHUNK,), jnp.int32),
          pltpu.VMEM((CHUNK,), jnp.int32),
          pltpu.VMEM((CHUNK, D), jnp.float32),
          pltpu.VMEM_SHARED((N_PAD, D), jnp.float32),
      ],
  )
  def agg_kernel(x_hbm, src_hbm, dst_hbm, out_hbm, sidx_v, didx_v, rows_v, acc):
    c = lax.axis_index("c")
    s = lax.axis_index("s")
    w = s * NC + c

    def zero_row(i, _):
      for j in range(D // 16):
        rows_v[i, pl.ds(j * 16, 16)] = jnp.zeros((16,), jnp.float32)
      return 0
    lax.fori_loop(0, CHUNK, zero_row, 0)
    for k in range(ROWS_PER_TILE // CHUNK):
      pltpu.sync_copy(rows_v, acc.at[pl.ds(s * ROWS_PER_TILE + k * CHUNK, CHUNK)])
    plsc.subcore_barrier()

    def body(j, _):
      base = w * E_TILE + j * CHUNK
      pltpu.sync_copy(src_hbm.at[pl.ds(base, CHUNK)], sidx_v)
      pltpu.sync_copy(dst_hbm.at[pl.ds(base, CHUNK)], didx_v)
      pltpu.sync_copy(x_hbm.at[sidx_v], rows_v)
      pltpu.sync_copy(rows_v, acc.at[didx_v], add=True)
      return 0
    lax.fori_loop(0, CHUNKS_PER_TILE, body, 0)
    plsc.subcore_barrier()

    pltpu.sync_copy(
        acc.at[pl.ds(s * ROWS_PER_TILE, ROWS_PER_TILE)],
        out_hbm.at[c, pl.ds(s * ROWS_PER_TILE, ROWS_PER_TILE)],
    )

  return agg_kernel(x, src_p, dst_p)


BLK = 1280


def _mm_body(p_ref, w_ref, b_ref, o_ref):
  s = p_ref[0] + p_ref[1]
  o_ref[...] = (
      jnp.dot(s, w_ref[...], preferred_element_type=jnp.float32) + b_ref[...]
  )


def _tc_matmul(parts, W, b2d):
  return pl.pallas_call(
      _mm_body,
      grid=(N_PAD // BLK,),
      in_specs=[
          pl.BlockSpec((NC, BLK, D), lambda i: (0, i, 0)),
          pl.BlockSpec((D, D), lambda i: (0, 0)),
          pl.BlockSpec((1, D), lambda i: (0, 0)),
      ],
      out_specs=pl.BlockSpec((BLK, D), lambda i: (i, 0)),
      out_shape=jax.ShapeDtypeStruct((N_PAD, D), jnp.float32),
  )(parts, W, b2d)


@jax.jit
def kernel(x, edge_index, W, b):
  src = edge_index[0].astype(jnp.int32)
  dst = edge_index[1].astype(jnp.int32)
  pad = E_PAD - N_EDGES
  src_p = jnp.concatenate([src, jnp.zeros((pad,), jnp.int32)])
  dst_p = jnp.concatenate([dst, jnp.full((pad,), N_NODES, jnp.int32)])
  parts = _sc_aggregate(x, src_p, dst_p)
  out = _tc_matmul(parts, W, b.reshape(1, D))
  return out[:N_NODES]
